# Initial kernel scaffold; baseline (speedup 1.0000x reference)
import functools

import jax
import jax.numpy as jnp
from jax import lax
from jax.experimental import pallas as pl
from jax.experimental.pallas import tpu as pltpu
from jax.experimental.pallas import tpu_sc as plsc

_MIN_NORM = 1e-15
_BN = 1024
_CH = 128
_NW = 32


def _artanh(x):
    x = jnp.clip(x, -1.0 + 1e-6, 1.0 - 1e-6)
    return 0.5 * jnp.log((1.0 + x) / (1.0 - x))


def _norm(x):
    return jnp.maximum(jnp.sqrt(jnp.sum(x * x, axis=-1, keepdims=True)),
                       _MIN_NORM)


def _proj(x):
    norm = _norm(x)
    maxnorm = 1.0 - 4e-3
    return jnp.where(norm > maxnorm, x / norm * maxnorm, x)


def _expmap0(u):
    un = _norm(u)
    return jnp.tanh(un) * u / un


def _logmap0(p):
    pn = _norm(p)
    return p / pn * _artanh(pn)


def _mobius_add(x, y):
    x2 = jnp.sum(x * x, -1, keepdims=True)
    y2 = jnp.sum(y * y, -1, keepdims=True)
    xy = jnp.sum(x * y, -1, keepdims=True)
    num = (1.0 + 2.0 * xy + y2) * x + (1.0 - x2) * y
    den = 1.0 + 2.0 * xy + x2 * y2
    return num / jnp.maximum(den, _MIN_NORM)


def _mobius_matvec(h, wt):
    xn = _norm(h)
    mx = jnp.dot(h, wt, preferred_element_type=jnp.float32)
    mxn = _norm(mx)
    res = jnp.tanh(mxn / xn * _artanh(xn)) * mx / mxn
    cond = jnp.all(mx == 0.0, axis=-1, keepdims=True)
    return jnp.where(cond, jnp.zeros_like(res), res)


def _post_agg(agg):
    h2 = _proj(_expmap0(agg))
    xt2 = jax.nn.relu(_logmap0(h2))
    return _proj(_expmap0(xt2))


def _hyp_linear_xt(h, wt, bvec):
    mv = _proj(_mobius_matvec(h, wt))
    hb = _proj(_expmap0(bvec))
    res = _proj(_mobius_add(mv, hb))
    return _logmap0(res)


def _tc0_body(x_ref, wt_ref, b_ref, out_ref):
    h = _proj(_expmap0(x_ref[...]))
    out_ref[...] = _hyp_linear_xt(h, wt_ref[...], b_ref[...])


def _tcmid_body(p_ref, wt_ref, b_ref, out_ref):
    h = _post_agg(p_ref[0] + p_ref[1])
    out_ref[...] = _hyp_linear_xt(h, wt_ref[...], b_ref[...])


def _tcfin_body(p_ref, out_ref):
    out_ref[...] = _post_agg(p_ref[0] + p_ref[1])


def _tc0(x, w0t, b0):
    n, in_dim = x.shape
    h_dim = w0t.shape[1]
    return pl.pallas_call(
        _tc0_body,
        grid=(n // _BN,),
        in_specs=[
            pl.BlockSpec((_BN, in_dim), lambda i: (i, 0)),
            pl.BlockSpec((in_dim, h_dim), lambda i: (0, 0)),
            pl.BlockSpec((1, h_dim), lambda i: (0, 0)),
        ],
        out_specs=pl.BlockSpec((_BN, h_dim), lambda i: (i, 0)),
        out_shape=jax.ShapeDtypeStruct((n, h_dim), jnp.float32),
    )(x, w0t, b0.reshape(1, -1))


def _tcmid(p, wti, bi):
    _, n, h_dim = p.shape
    return pl.pallas_call(
        _tcmid_body,
        grid=(n // _BN,),
        in_specs=[
            pl.BlockSpec((2, _BN, h_dim), lambda i: (0, i, 0)),
            pl.BlockSpec((h_dim, h_dim), lambda i: (0, 0)),
            pl.BlockSpec((1, h_dim), lambda i: (0, 0)),
        ],
        out_specs=pl.BlockSpec((_BN, h_dim), lambda i: (i, 0)),
        out_shape=jax.ShapeDtypeStruct((n, h_dim), jnp.float32),
    )(p, wti, bi.reshape(1, -1))


def _tcfin(p):
    _, n, h_dim = p.shape
    return pl.pallas_call(
        _tcfin_body,
        grid=(n // _BN,),
        in_specs=[pl.BlockSpec((2, _BN, h_dim), lambda i: (0, i, 0))],
        out_specs=pl.BlockSpec((_BN, h_dim), lambda i: (i, 0)),
        out_shape=jax.ShapeDtypeStruct((n, h_dim), jnp.float32),
    )(p)


@functools.lru_cache(maxsize=None)
def _make_agg(n_pad, h_dim, k_chunks):
    rows_per_tile = n_pad // 16
    mesh = plsc.VectorSubcoreMesh(core_axis_name="c", subcore_axis_name="s")

    @functools.partial(
        pl.kernel,
        out_type=jax.ShapeDtypeStruct((2, n_pad, h_dim), jnp.float32),
        mesh=mesh,
        scratch_types=[
            pltpu.VMEM((k_chunks, _CH), jnp.int32),
            pltpu.VMEM((k_chunks, _CH), jnp.int32),
            pltpu.VMEM((_CH, h_dim), jnp.float32),
            pltpu.VMEM_SHARED((n_pad, h_dim), jnp.float32),
            pltpu.SemaphoreType.DMA,
        ],
    )
    def agg(xt_hbm, src_hbm, dst_hbm, zeros_hbm, out_hbm,
            sidx_v, didx_v, rows_v, acc_sh, sem):
        cid = lax.axis_index("c")
        sid = lax.axis_index("s")
        wid = sid * 2 + cid
        r0 = sid * rows_per_tile
        pltpu.sync_copy(zeros_hbm.at[pl.ds(r0, rows_per_tile)],
                        acc_sh.at[pl.ds(r0, rows_per_tile)])
        pltpu.sync_copy(src_hbm.at[wid], sidx_v)
        pltpu.sync_copy(dst_hbm.at[wid], didx_v)
        plsc.subcore_barrier()

        def body(j, carry):
            pltpu.async_copy(xt_hbm.at[sidx_v.at[j]], rows_v, sem).wait()
            pltpu.sync_copy(rows_v, acc_sh.at[didx_v.at[j]], add=True)
            return carry

        lax.fori_loop(0, k_chunks, body, 0)
        plsc.subcore_barrier()
        pltpu.sync_copy(acc_sh.at[pl.ds(r0, rows_per_tile)],
                        out_hbm.at[cid, pl.ds(r0, rows_per_tile)])

    return agg


def kernel(x, edge_index, W0, b0, W, b):
    n, in_dim = x.shape
    h_dim = W0.shape[0]
    e = edge_index.shape[1]

    n_pad = -(-n // _BN) * _BN
    e_pad = -(-e // (_NW * _CH)) * (_NW * _CH)
    k_chunks = e_pad // (_NW * _CH)

    x_p = jnp.zeros((n_pad, in_dim), jnp.float32).at[:n].set(x)
    ei = edge_index.astype(jnp.int32)
    src = jnp.concatenate(
        [ei[0], jnp.zeros((e_pad - e,), jnp.int32)]).reshape(_NW, k_chunks, _CH)
    dst = jnp.concatenate(
        [ei[1], jnp.full((e_pad - e,), n, jnp.int32)]).reshape(_NW, k_chunks, _CH)
    zeros = jnp.zeros((n_pad, h_dim), jnp.float32)

    w0t = W0.T
    wt = W.transpose(0, 2, 1)
    agg_fn = _make_agg(n_pad, h_dim, k_chunks)

    xt = _tc0(x_p, w0t, b0)
    p = agg_fn(xt, src, dst, zeros)

    def body(carry, wb):
        wti, bi = wb
        xt_i = _tcmid(carry, wti, bi)
        return agg_fn(xt_i, src, dst, zeros), None

    p, _ = lax.scan(body, p, (wt, b))
    return _tcfin(p)[:n]


# speedup vs baseline: 4.5312x; 4.5312x over previous
import functools

import jax
import jax.numpy as jnp
from jax import lax
from jax.experimental import pallas as pl
from jax.experimental.pallas import tpu as pltpu
from jax.experimental.pallas import tpu_sc as plsc

_MIN_NORM = 1e-15
_BN = 1024
_CH = 128
_NW = 32


def _artanh(x):
    x = jnp.clip(x, -1.0 + 1e-6, 1.0 - 1e-6)
    return 0.5 * (jnp.log1p(x) - jnp.log1p(-x))


def _norm(x):
    return jnp.maximum(jnp.sqrt(jnp.sum(x * x, axis=-1, keepdims=True)),
                       _MIN_NORM)


def _proj(x):
    norm = _norm(x)
    maxnorm = 1.0 - 4e-3
    return jnp.where(norm > maxnorm, x / norm * maxnorm, x)


def _expmap0(u):
    un = _norm(u)
    return jnp.tanh(un) * u / un


def _logmap0(p):
    pn = _norm(p)
    return p / pn * _artanh(pn)


def _mobius_add(x, y):
    x2 = jnp.sum(x * x, -1, keepdims=True)
    y2 = jnp.sum(y * y, -1, keepdims=True)
    xy = jnp.sum(x * y, -1, keepdims=True)
    num = (1.0 + 2.0 * xy + y2) * x + (1.0 - x2) * y
    den = 1.0 + 2.0 * xy + x2 * y2
    return num / jnp.maximum(den, _MIN_NORM)


def _mobius_matvec(h, wt):
    xn = _norm(h)
    mx = jnp.dot(h, wt, preferred_element_type=jnp.float32)
    mxn = _norm(mx)
    res = jnp.tanh(mxn / xn * _artanh(xn)) * mx / mxn
    cond = jnp.all(mx == 0.0, axis=-1, keepdims=True)
    return jnp.where(cond, jnp.zeros_like(res), res)


def _post_agg(agg):
    h2 = _proj(_expmap0(agg))
    xt2 = jax.nn.relu(_logmap0(h2))
    return _proj(_expmap0(xt2))


def _hyp_linear_xt(h, wt, bvec):
    mv = _proj(_mobius_matvec(h, wt))
    hb = _proj(_expmap0(bvec))
    res = _proj(_mobius_add(mv, hb))
    return _logmap0(res)


def _tc0_body(x_ref, wt_ref, b_ref, out_ref):
    h = _proj(_expmap0(x_ref[...]))
    out_ref[...] = _hyp_linear_xt(h, wt_ref[...], b_ref[...])


def _tcmid_body(p_ref, wt_ref, b_ref, out_ref):
    h = _post_agg(p_ref[0] + p_ref[1])
    out_ref[...] = _hyp_linear_xt(h, wt_ref[...], b_ref[...])


def _tcfin_body(p_ref, out_ref):
    out_ref[...] = _post_agg(p_ref[0] + p_ref[1])


def _tc0(x, w0t, b0):
    n, in_dim = x.shape
    h_dim = w0t.shape[1]
    return pl.pallas_call(
        _tc0_body,
        grid=(n // _BN,),
        in_specs=[
            pl.BlockSpec((_BN, in_dim), lambda i: (i, 0)),
            pl.BlockSpec((in_dim, h_dim), lambda i: (0, 0)),
            pl.BlockSpec((1, h_dim), lambda i: (0, 0)),
        ],
        out_specs=pl.BlockSpec((_BN, h_dim), lambda i: (i, 0)),
        out_shape=jax.ShapeDtypeStruct((n, h_dim), jnp.float32),
    )(x, w0t, b0.reshape(1, -1))


def _tcmid(p, wti, bi):
    _, n, h_dim = p.shape
    return pl.pallas_call(
        _tcmid_body,
        grid=(n // _BN,),
        in_specs=[
            pl.BlockSpec((2, _BN, h_dim), lambda i: (0, i, 0)),
            pl.BlockSpec((h_dim, h_dim), lambda i: (0, 0)),
            pl.BlockSpec((1, h_dim), lambda i: (0, 0)),
        ],
        out_specs=pl.BlockSpec((_BN, h_dim), lambda i: (i, 0)),
        out_shape=jax.ShapeDtypeStruct((n, h_dim), jnp.float32),
    )(p, wti, bi.reshape(1, -1))


def _tcfin(p):
    _, n, h_dim = p.shape
    return pl.pallas_call(
        _tcfin_body,
        grid=(n // _BN,),
        in_specs=[pl.BlockSpec((2, _BN, h_dim), lambda i: (0, i, 0))],
        out_specs=pl.BlockSpec((_BN, h_dim), lambda i: (i, 0)),
        out_shape=jax.ShapeDtypeStruct((n, h_dim), jnp.float32),
    )(p)


@functools.lru_cache(maxsize=None)
def _make_agg(n_pad, h_dim, k_chunks):
    rows_per_tile = n_pad // 16
    mesh = plsc.VectorSubcoreMesh(core_axis_name="c", subcore_axis_name="s")

    @functools.partial(
        pl.kernel,
        out_type=jax.ShapeDtypeStruct((2, n_pad, h_dim), jnp.float32),
        mesh=mesh,
        scratch_types=[
            pltpu.VMEM((k_chunks, _CH), jnp.int32),
            pltpu.VMEM((k_chunks, _CH), jnp.int32),
            pltpu.VMEM((_CH, h_dim), jnp.float32),
            pltpu.VMEM_SHARED((n_pad, h_dim), jnp.float32),
            pltpu.SemaphoreType.DMA,
        ],
        compiler_params=pltpu.CompilerParams(use_tc_tiling_on_sc=False),
    )
    def agg(xt_hbm, src_hbm, dst_hbm, zeros_hbm, out_hbm,
            sidx_v, didx_v, rows_v, acc_sh, sem):
        cid = lax.axis_index("c")
        sid = lax.axis_index("s")
        wid = sid * 2 + cid
        r0 = sid * rows_per_tile
        pltpu.sync_copy(zeros_hbm.at[pl.ds(r0, rows_per_tile)],
                        acc_sh.at[pl.ds(r0, rows_per_tile)])
        pltpu.sync_copy(src_hbm.at[wid], sidx_v)
        pltpu.sync_copy(dst_hbm.at[wid], didx_v)
        plsc.subcore_barrier()

        def body(j, carry):
            pltpu.async_copy(xt_hbm.at[sidx_v.at[j]], rows_v, sem).wait()
            pltpu.sync_copy(rows_v, acc_sh.at[didx_v.at[j]], add=True)
            return carry

        lax.fori_loop(0, k_chunks, body, 0)
        plsc.subcore_barrier()
        pltpu.sync_copy(acc_sh.at[pl.ds(r0, rows_per_tile)],
                        out_hbm.at[cid, pl.ds(r0, rows_per_tile)])

    return agg


def kernel(x, edge_index, W0, b0, W, b):
    n, in_dim = x.shape
    h_dim = W0.shape[0]
    e = edge_index.shape[1]

    n_pad = -(-n // _BN) * _BN
    e_pad = -(-e // (_NW * _CH)) * (_NW * _CH)
    k_chunks = e_pad // (_NW * _CH)

    x_p = jnp.zeros((n_pad, in_dim), jnp.float32).at[:n].set(x)
    ei = edge_index.astype(jnp.int32)
    src = jnp.concatenate(
        [ei[0], jnp.zeros((e_pad - e,), jnp.int32)]).reshape(_NW, k_chunks, _CH)
    dst = jnp.concatenate(
        [ei[1], jnp.full((e_pad - e,), n, jnp.int32)]).reshape(_NW, k_chunks, _CH)
    zeros = jnp.zeros((n_pad, h_dim), jnp.float32)

    w0t = W0.T
    wt = W.transpose(0, 2, 1)
    agg_fn = _make_agg(n_pad, h_dim, k_chunks)

    xt = _tc0(x_p, w0t, b0)
    p = agg_fn(xt, src, dst, zeros)

    def body(carry, wb):
        wti, bi = wb
        xt_i = _tcmid(carry, wti, bi)
        return agg_fn(xt_i, src, dst, zeros), None

    p, _ = lax.scan(body, p, (wt, b))
    return _tcfin(p)[:n]


# speedup vs baseline: 5.2090x; 1.1496x over previous
import functools

import jax
import jax.numpy as jnp
from jax import lax
from jax.experimental import pallas as pl
from jax.experimental.pallas import tpu as pltpu
from jax.experimental.pallas import tpu_sc as plsc

_MIN_NORM = 1e-15
_BN = 1024
_CH = 128
_NW = 32


def _artanh(x):
    x = jnp.clip(x, -1.0 + 1e-6, 1.0 - 1e-6)
    return 0.5 * (jnp.log1p(x) - jnp.log1p(-x))


def _norm(x):
    return jnp.maximum(jnp.sqrt(jnp.sum(x * x, axis=-1, keepdims=True)),
                       _MIN_NORM)


def _proj(x):
    norm = _norm(x)
    maxnorm = 1.0 - 4e-3
    return jnp.where(norm > maxnorm, x / norm * maxnorm, x)


def _expmap0(u):
    un = _norm(u)
    return jnp.tanh(un) * u / un


def _logmap0(p):
    pn = _norm(p)
    return p / pn * _artanh(pn)


def _mobius_add(x, y):
    x2 = jnp.sum(x * x, -1, keepdims=True)
    y2 = jnp.sum(y * y, -1, keepdims=True)
    xy = jnp.sum(x * y, -1, keepdims=True)
    num = (1.0 + 2.0 * xy + y2) * x + (1.0 - x2) * y
    den = 1.0 + 2.0 * xy + x2 * y2
    return num / jnp.maximum(den, _MIN_NORM)


def _mobius_matvec(h, wt):
    xn = _norm(h)
    mx = jnp.dot(h, wt, preferred_element_type=jnp.float32)
    mxn = _norm(mx)
    res = jnp.tanh(mxn / xn * _artanh(xn)) * mx / mxn
    cond = jnp.all(mx == 0.0, axis=-1, keepdims=True)
    return jnp.where(cond, jnp.zeros_like(res), res)


def _post_agg(agg):
    h2 = _proj(_expmap0(agg))
    xt2 = jax.nn.relu(_logmap0(h2))
    return _proj(_expmap0(xt2))


def _hyp_linear_xt(h, wt, bvec):
    mv = _proj(_mobius_matvec(h, wt))
    hb = _proj(_expmap0(bvec))
    res = _proj(_mobius_add(mv, hb))
    return _logmap0(res)


def _tc0_body(x_ref, wt_ref, b_ref, out_ref):
    h = _proj(_expmap0(x_ref[...]))
    out_ref[...] = _hyp_linear_xt(h, wt_ref[...], b_ref[...])


def _tcmid_body(p_ref, wt_ref, b_ref, out_ref):
    h = _post_agg(p_ref[0] + p_ref[1])
    out_ref[...] = _hyp_linear_xt(h, wt_ref[...], b_ref[...])


def _tcfin_body(p_ref, out_ref):
    out_ref[...] = _post_agg(p_ref[0] + p_ref[1])


def _tc0(x, w0t, b0):
    n, in_dim = x.shape
    h_dim = w0t.shape[1]
    return pl.pallas_call(
        _tc0_body,
        grid=(n // _BN,),
        in_specs=[
            pl.BlockSpec((_BN, in_dim), lambda i: (i, 0)),
            pl.BlockSpec((in_dim, h_dim), lambda i: (0, 0)),
            pl.BlockSpec((1, h_dim), lambda i: (0, 0)),
        ],
        out_specs=pl.BlockSpec((_BN, h_dim), lambda i: (i, 0)),
        out_shape=jax.ShapeDtypeStruct((n, h_dim), jnp.float32),
    )(x, w0t, b0.reshape(1, -1))


def _tcmid(p, wti, bi):
    _, n, h_dim = p.shape
    return pl.pallas_call(
        _tcmid_body,
        grid=(n // _BN,),
        in_specs=[
            pl.BlockSpec((2, _BN, h_dim), lambda i: (0, i, 0)),
            pl.BlockSpec((h_dim, h_dim), lambda i: (0, 0)),
            pl.BlockSpec((1, h_dim), lambda i: (0, 0)),
        ],
        out_specs=pl.BlockSpec((_BN, h_dim), lambda i: (i, 0)),
        out_shape=jax.ShapeDtypeStruct((n, h_dim), jnp.float32),
    )(p, wti, bi.reshape(1, -1))


def _tcfin(p):
    _, n, h_dim = p.shape
    return pl.pallas_call(
        _tcfin_body,
        grid=(n // _BN,),
        in_specs=[pl.BlockSpec((2, _BN, h_dim), lambda i: (0, i, 0))],
        out_specs=pl.BlockSpec((_BN, h_dim), lambda i: (i, 0)),
        out_shape=jax.ShapeDtypeStruct((n, h_dim), jnp.float32),
    )(p)


_NBUF = 6
_PF = 4


@functools.lru_cache(maxsize=None)
def _make_agg(n_pad, h_dim, k_chunks):
    rows_per_tile = n_pad // 16
    mesh = plsc.VectorSubcoreMesh(core_axis_name="c", subcore_axis_name="s")

    @functools.partial(
        pl.kernel,
        out_type=jax.ShapeDtypeStruct((2, n_pad, h_dim), jnp.float32),
        mesh=mesh,
        scratch_types=[
            pltpu.VMEM((k_chunks, _CH), jnp.int32),
            pltpu.VMEM((k_chunks, _CH), jnp.int32),
            pltpu.VMEM((_NBUF, _CH, h_dim), jnp.float32),
            pltpu.VMEM_SHARED((n_pad, h_dim), jnp.float32),
            pltpu.SemaphoreType.DMA((_NBUF,)),
            pltpu.SemaphoreType.DMA((_NBUF,)),
        ],
        compiler_params=pltpu.CompilerParams(use_tc_tiling_on_sc=False),
    )
    def agg(xt_hbm, src_hbm, dst_hbm, zeros_hbm, out_hbm,
            sidx_v, didx_v, rows_v, acc_sh, gsem, ssem):
        cid = lax.axis_index("c")
        sid = lax.axis_index("s")
        wid = sid * 2 + cid
        r0 = sid * rows_per_tile
        pltpu.sync_copy(zeros_hbm.at[pl.ds(r0, rows_per_tile)],
                        acc_sh.at[pl.ds(r0, rows_per_tile)])
        pltpu.sync_copy(src_hbm.at[wid], sidx_v)
        pltpu.sync_copy(dst_hbm.at[wid], didx_v)
        plsc.subcore_barrier()

        g = [None] * _NBUF
        s = [None] * _NBUF

        def fire_gather(j):
            b = j % _NBUF
            if s[b] is not None:
                s[b].wait()
                s[b] = None
            g[b] = pltpu.async_copy(
                xt_hbm.at[sidx_v.at[j]], rows_v.at[b], gsem.at[b])

        for j in range(min(_PF, k_chunks)):
            fire_gather(j)
        for j in range(k_chunks):
            b = j % _NBUF
            g[b].wait()
            s[b] = pltpu.async_copy(
                rows_v.at[b], acc_sh.at[didx_v.at[j]], ssem.at[b], add=True)
            if j + _PF < k_chunks:
                fire_gather(j + _PF)
        for b in range(_NBUF):
            if s[b] is not None:
                s[b].wait()

        plsc.subcore_barrier()
        pltpu.sync_copy(acc_sh.at[pl.ds(r0, rows_per_tile)],
                        out_hbm.at[cid, pl.ds(r0, rows_per_tile)])

    return agg


def kernel(x, edge_index, W0, b0, W, b):
    n, in_dim = x.shape
    h_dim = W0.shape[0]
    e = edge_index.shape[1]

    n_pad = -(-n // _BN) * _BN
    e_pad = -(-e // (_NW * _CH)) * (_NW * _CH)
    k_chunks = e_pad // (_NW * _CH)

    x_p = jnp.zeros((n_pad, in_dim), jnp.float32).at[:n].set(x)
    ei = edge_index.astype(jnp.int32)
    src = jnp.concatenate(
        [ei[0], jnp.zeros((e_pad - e,), jnp.int32)]).reshape(_NW, k_chunks, _CH)
    dst = jnp.concatenate(
        [ei[1], jnp.full((e_pad - e,), n, jnp.int32)]).reshape(_NW, k_chunks, _CH)
    zeros = jnp.zeros((n_pad, h_dim), jnp.float32)

    w0t = W0.T
    wt = W.transpose(0, 2, 1)
    agg_fn = _make_agg(n_pad, h_dim, k_chunks)

    xt = _tc0(x_p, w0t, b0)
    p = agg_fn(xt, src, dst, zeros)

    def body(carry, wb):
        wti, bi = wb
        xt_i = _tcmid(carry, wti, bi)
        return agg_fn(xt_i, src, dst, zeros), None

    p, _ = lax.scan(body, p, (wt, b))
    return _tcfin(p)[:n]


# speedup vs baseline: 5.2113x; 1.0004x over previous
import functools

import jax
import jax.numpy as jnp
from jax import lax
from jax.experimental import pallas as pl
from jax.experimental.pallas import tpu as pltpu
from jax.experimental.pallas import tpu_sc as plsc

_MIN_NORM = 1e-15
_BN = 1024
_CH = 128
_NW = 32


def _artanh(x):
    x = jnp.clip(x, -1.0 + 1e-6, 1.0 - 1e-6)
    return 0.5 * (jnp.log1p(x) - jnp.log1p(-x))


def _norm(x):
    return jnp.maximum(jnp.sqrt(jnp.sum(x * x, axis=-1, keepdims=True)),
                       _MIN_NORM)


def _proj(x):
    norm = _norm(x)
    maxnorm = 1.0 - 4e-3
    return jnp.where(norm > maxnorm, x / norm * maxnorm, x)


def _expmap0(u):
    un = _norm(u)
    return jnp.tanh(un) * u / un


def _logmap0(p):
    pn = _norm(p)
    return p / pn * _artanh(pn)


def _mobius_add(x, y):
    x2 = jnp.sum(x * x, -1, keepdims=True)
    y2 = jnp.sum(y * y, -1, keepdims=True)
    xy = jnp.sum(x * y, -1, keepdims=True)
    num = (1.0 + 2.0 * xy + y2) * x + (1.0 - x2) * y
    den = 1.0 + 2.0 * xy + x2 * y2
    return num / jnp.maximum(den, _MIN_NORM)


def _mobius_matvec(h, wt):
    xn = _norm(h)
    mx = jnp.dot(h, wt, preferred_element_type=jnp.float32)
    mxn = _norm(mx)
    res = jnp.tanh(mxn / xn * _artanh(xn)) * mx / mxn
    cond = jnp.all(mx == 0.0, axis=-1, keepdims=True)
    return jnp.where(cond, jnp.zeros_like(res), res)


def _post_agg(agg):
    h2 = _proj(_expmap0(agg))
    xt2 = jax.nn.relu(_logmap0(h2))
    return _proj(_expmap0(xt2))


def _hyp_linear_xt(h, wt, bvec):
    mv = _proj(_mobius_matvec(h, wt))
    hb = _proj(_expmap0(bvec))
    res = _proj(_mobius_add(mv, hb))
    return _logmap0(res)


def _tc0_body(x_ref, wt_ref, b_ref, out_ref):
    h = _proj(_expmap0(x_ref[...]))
    out_ref[...] = _hyp_linear_xt(h, wt_ref[...], b_ref[...])


def _tcmid_body(p_ref, wt_ref, b_ref, out_ref):
    h = _post_agg(p_ref[0] + p_ref[1])
    out_ref[...] = _hyp_linear_xt(h, wt_ref[...], b_ref[...])


def _tcfin_body(p_ref, out_ref):
    out_ref[...] = _post_agg(p_ref[0] + p_ref[1])


def _tc0(x, w0t, b0):
    n, in_dim = x.shape
    h_dim = w0t.shape[1]
    return pl.pallas_call(
        _tc0_body,
        grid=(n // _BN,),
        in_specs=[
            pl.BlockSpec((_BN, in_dim), lambda i: (i, 0)),
            pl.BlockSpec((in_dim, h_dim), lambda i: (0, 0)),
            pl.BlockSpec((1, h_dim), lambda i: (0, 0)),
        ],
        out_specs=pl.BlockSpec((_BN, h_dim), lambda i: (i, 0)),
        out_shape=jax.ShapeDtypeStruct((n, h_dim), jnp.float32),
    )(x, w0t, b0.reshape(1, -1))


def _tcmid(p, wti, bi):
    _, n, h_dim = p.shape
    return pl.pallas_call(
        _tcmid_body,
        grid=(n // _BN,),
        in_specs=[
            pl.BlockSpec((2, _BN, h_dim), lambda i: (0, i, 0)),
            pl.BlockSpec((h_dim, h_dim), lambda i: (0, 0)),
            pl.BlockSpec((1, h_dim), lambda i: (0, 0)),
        ],
        out_specs=pl.BlockSpec((_BN, h_dim), lambda i: (i, 0)),
        out_shape=jax.ShapeDtypeStruct((n, h_dim), jnp.float32),
    )(p, wti, bi.reshape(1, -1))


def _tcfin(p):
    _, n, h_dim = p.shape
    return pl.pallas_call(
        _tcfin_body,
        grid=(n // _BN,),
        in_specs=[pl.BlockSpec((2, _BN, h_dim), lambda i: (0, i, 0))],
        out_specs=pl.BlockSpec((_BN, h_dim), lambda i: (i, 0)),
        out_shape=jax.ShapeDtypeStruct((n, h_dim), jnp.float32),
    )(p)


_NBUF = 6
_PF = 4


@functools.lru_cache(maxsize=None)
def _make_agg(n_pad, h_dim, k_chunks):
    rows_per_tile = n_pad // 16
    mesh = plsc.VectorSubcoreMesh(core_axis_name="c", subcore_axis_name="s")

    @functools.partial(
        pl.kernel,
        out_type=jax.ShapeDtypeStruct((2, n_pad, h_dim), jnp.float32),
        mesh=mesh,
        scratch_types=[
            pltpu.VMEM((k_chunks, _CH), jnp.int32),
            pltpu.VMEM((k_chunks, _CH), jnp.int32),
            pltpu.VMEM((_NBUF, _CH, h_dim), jnp.float32),
            pltpu.VMEM_SHARED((n_pad, h_dim), jnp.float32),
        ] + [pltpu.SemaphoreType.DMA] * (2 * _NBUF),
        compiler_params=pltpu.CompilerParams(use_tc_tiling_on_sc=False),
    )
    def agg(xt_hbm, src_hbm, dst_hbm, zeros_hbm, out_hbm,
            sidx_v, didx_v, rows_v, acc_sh, *sems):
        gsem = sems[:_NBUF]
        ssem = sems[_NBUF:]
        cid = lax.axis_index("c")
        sid = lax.axis_index("s")
        wid = sid * 2 + cid
        r0 = sid * rows_per_tile
        pltpu.sync_copy(zeros_hbm.at[pl.ds(r0, rows_per_tile)],
                        acc_sh.at[pl.ds(r0, rows_per_tile)])
        pltpu.sync_copy(src_hbm.at[wid], sidx_v)
        pltpu.sync_copy(dst_hbm.at[wid], didx_v)
        plsc.subcore_barrier()

        g = [None] * _NBUF
        s = [None] * _NBUF

        def fire_gather(j):
            b = j % _NBUF
            if s[b] is not None:
                s[b].wait()
                s[b] = None
            g[b] = pltpu.async_copy(
                xt_hbm.at[sidx_v.at[j]], rows_v.at[b], gsem[b])

        for j in range(min(_PF, k_chunks)):
            fire_gather(j)
        for j in range(k_chunks):
            b = j % _NBUF
            g[b].wait()
            s[b] = pltpu.async_copy(
                rows_v.at[b], acc_sh.at[didx_v.at[j]], ssem[b], add=True)
            if j + _PF < k_chunks:
                fire_gather(j + _PF)
        for b in range(_NBUF):
            if s[b] is not None:
                s[b].wait()

        plsc.subcore_barrier()
        pltpu.sync_copy(acc_sh.at[pl.ds(r0, rows_per_tile)],
                        out_hbm.at[cid, pl.ds(r0, rows_per_tile)])

    return agg


def kernel(x, edge_index, W0, b0, W, b):
    n, in_dim = x.shape
    h_dim = W0.shape[0]
    e = edge_index.shape[1]

    n_pad = -(-n // _BN) * _BN
    e_pad = -(-e // (_NW * _CH)) * (_NW * _CH)
    k_chunks = e_pad // (_NW * _CH)

    x_p = jnp.zeros((n_pad, in_dim), jnp.float32).at[:n].set(x)
    ei = edge_index.astype(jnp.int32)
    src = jnp.concatenate(
        [ei[0], jnp.zeros((e_pad - e,), jnp.int32)]).reshape(_NW, k_chunks, _CH)
    dst = jnp.concatenate(
        [ei[1], jnp.full((e_pad - e,), n, jnp.int32)]).reshape(_NW, k_chunks, _CH)
    zeros = jnp.zeros((n_pad, h_dim), jnp.float32)

    w0t = W0.T
    wt = W.transpose(0, 2, 1)
    agg_fn = _make_agg(n_pad, h_dim, k_chunks)

    xt = _tc0(x_p, w0t, b0)
    p = agg_fn(xt, src, dst, zeros)

    def body(carry, wb):
        wti, bi = wb
        xt_i = _tcmid(carry, wti, bi)
        return agg_fn(xt_i, src, dst, zeros), None

    p, _ = lax.scan(body, p, (wt, b))
    return _tcfin(p)[:n]
